# Initial kernel scaffold; baseline (speedup 1.0000x reference)
import jax
import jax.numpy as jnp
from jax.experimental import pallas as pl


def kernel(x, batch, W, b):
    raise NotImplementedError("write your pallas kernel here")


# speedup vs baseline: 2.6935x; 2.6935x over previous
import functools

import jax
import jax.numpy as jnp
from jax import lax
from jax.experimental import pallas as pl
from jax.experimental.pallas import tpu as pltpu
from jax.experimental.pallas import tpu_sc as plsc

_S = 512
_R = 256
_NW = 32
_SEG_W = _S // _NW


def _sc_readout(x, batch_i32, seg_starts, w, b):
    N, D = x.shape
    K = D // 16

    mesh = plsc.VectorSubcoreMesh(core_axis_name="c", subcore_axis_name="s")

    @functools.partial(
        pl.kernel,
        out_type=jax.ShapeDtypeStruct((_S, 2 * D), jnp.float32),
        mesh=mesh,
        scratch_types=[
            pltpu.VMEM((2, _R, D), jnp.float32),
            pltpu.VMEM((704,), jnp.int32),
            pltpu.VMEM((48,), jnp.int32),
            pltpu.VMEM((D,), jnp.float32),
            pltpu.VMEM((16,), jnp.float32),
            pltpu.VMEM((2 * D,), jnp.float32),
            pltpu.SemaphoreType.DMA,
            pltpu.SemaphoreType.DMA,
            pltpu.SemaphoreType.DMA,
        ],
        compiler_params=pltpu.CompilerParams(needs_layout_passes=False),
    )
    def body(x_hbm, b_hbm, ss_hbm, w_hbm, bias_hbm, out_hbm,
             xbuf, bbuf, ss_v, w_v, bias_ref, stage, sem_x, sem_b, sem_o):
        wid = lax.axis_index("s") * 2 + lax.axis_index("c")
        seg0 = wid * _SEG_W
        pltpu.sync_copy(ss_hbm.at[pl.ds(seg0, 32)], ss_v.at[pl.ds(0, 32)])
        pltpu.sync_copy(w_hbm, w_v)
        pltpu.sync_copy(bias_hbm, bias_ref)
        bias_v = bias_ref[pl.ds(0, 16)]
        wv = [w_v[pl.ds(16 * k, 16)] for k in range(K)]
        lo = ss_v[pl.ds(0, 16)][0]
        hi = ss_v[pl.ds(_SEG_W, 16)][0]

        zeros = jnp.zeros((16,), jnp.float32)
        ninf = jnp.full((16,), -jnp.inf, jnp.float32)

        for k in range(K):
            stage[pl.ds(16 * k, 16)] = zeros
            stage[pl.ds(D + 16 * k, 16)] = ninf
        for j in range(_SEG_W):
            pltpu.make_async_copy(stage, out_hbm.at[seg0 + j], sem_o).start()
        for j in range(_SEG_W):
            pltpu.make_async_copy(stage, out_hbm.at[seg0 + j], sem_o).wait()

        lo_al = (lo // 8) * 8
        n_chunks = jnp.maximum((hi - lo_al + _R - 1) // _R, 1)

        def x_start(t):
            return pl.multiple_of(jnp.minimum(lo_al + t * _R, N - _R), 8)

        def b_start(t):
            return pl.multiple_of(jnp.minimum(lo_al + t * _R, N - 320), 8)

        def x_dma(t):
            return pltpu.make_async_copy(
                x_hbm.at[pl.ds(x_start(t), _R)], xbuf.at[t % 2], sem_x)

        def b_dma(t):
            return pltpu.make_async_copy(
                b_hbm.at[pl.ds(b_start(t), 320)],
                bbuf.at[pl.ds((t % 2) * 320, 320)], sem_b)

        x_dma(0).start()
        b_dma(0).start()

        def chunk_body(t, accs):
            g0 = jnp.maximum(lo, lo_al + t * _R)
            g1 = jnp.minimum(lo_al + (t + 1) * _R, hi)
            start = x_start(t)
            bstart = b_start(t)
            x_dma(t).wait()
            b_dma(t).wait()

            @pl.when(t + 1 < n_chunks)
            def _():
                x_dma(t + 1).start()
                b_dma(t + 1).start()

            xs = xbuf.at[t % 2]
            bbase = (t % 2) * 320

            def row_body(r, a):
                bi = r - start
                ids = bbuf[pl.ds(bbase + r - bstart, 16)]
                cur = ids[0]
                flush = (ids[1] != cur) | (r == hi - 1)
                xk = [xs[bi, pl.ds(16 * k, 16)] for k in range(K)]
                pv = xk[0] * wv[0]
                for k in range(1, K):
                    pv = pv + xk[k] * wv[k]
                tlog = jnp.broadcast_to(jnp.sum(pv), (16,)) + bias_v
                sc = 1.0 / (1.0 + jnp.exp(-tlog))
                sums = [a[k] + sc * xk[k] for k in range(K)]
                maxs = [jnp.maximum(a[K + k], xk[k]) for k in range(K)]

                @pl.when(flush)
                def _():
                    for k in range(K):
                        stage[pl.ds(16 * k, 16)] = sums[k]
                        stage[pl.ds(D + 16 * k, 16)] = maxs[k]
                    pltpu.sync_copy(stage, out_hbm.at[cur])

                out = tuple(jnp.where(flush, zeros, sums[k])
                            for k in range(K))
                out += tuple(jnp.where(flush, ninf, maxs[k])
                             for k in range(K))
                return out

            return lax.fori_loop(g0, g1, row_body, accs)

        init = (zeros,) * K + (ninf,) * K
        lax.fori_loop(0, n_chunks, chunk_body, init)

    return body(x, batch_i32, seg_starts, w, b)


def kernel(x, batch, W, b):
    N, D = x.shape
    bi32 = batch.astype(jnp.int32)
    ss = jnp.searchsorted(bi32, jnp.arange(_S + 1, dtype=jnp.int32),
                          side="left").astype(jnp.int32)
    ss_pad = jnp.concatenate(
        [ss, jnp.full((528 - _S - 1,), N, dtype=jnp.int32)])
    bp = jnp.full((16,), b[0], dtype=jnp.float32)
    return _sc_readout(x, bi32, ss_pad, W.reshape(D).astype(jnp.float32), bp)


# speedup vs baseline: 3.5578x; 1.3209x over previous
import functools

import jax
import jax.numpy as jnp
from jax import lax
from jax.experimental import pallas as pl
from jax.experimental.pallas import tpu as pltpu
from jax.experimental.pallas import tpu_sc as plsc

_S = 512
_R = 256
_NW = 32
_SEG_W = _S // _NW


def _sc_readout(x, batch_i32, seg_starts, w, b):
    N, D = x.shape
    K = D // 16

    mesh = plsc.VectorSubcoreMesh(core_axis_name="c", subcore_axis_name="s")

    @functools.partial(
        pl.kernel,
        out_type=jax.ShapeDtypeStruct((_S, 2 * D), jnp.float32),
        mesh=mesh,
        scratch_types=[
            pltpu.VMEM((2, _R + 8, D), jnp.float32),
            pltpu.VMEM((704,), jnp.int32),
            pltpu.VMEM((48,), jnp.int32),
            pltpu.VMEM((D,), jnp.float32),
            pltpu.VMEM((16,), jnp.float32),
            pltpu.VMEM((4 * D,), jnp.float32),
            pltpu.SemaphoreType.DMA,
            pltpu.SemaphoreType.DMA,
            pltpu.SemaphoreType.DMA,
        ],
        compiler_params=pltpu.CompilerParams(needs_layout_passes=False),
    )
    def body(x_hbm, b_hbm, ss_hbm, w_hbm, bias_hbm, out_hbm,
             xbuf, bbuf, ss_v, w_v, bias_ref, stage, sem_x, sem_b, sem_o):
        wid = lax.axis_index("s") * 2 + lax.axis_index("c")
        seg0 = wid * _SEG_W
        pltpu.sync_copy(ss_hbm.at[pl.ds(seg0, 32)], ss_v.at[pl.ds(0, 32)])
        pltpu.sync_copy(w_hbm, w_v)
        pltpu.sync_copy(bias_hbm, bias_ref)
        bias_v = bias_ref[pl.ds(0, 16)]
        wv = [w_v[pl.ds(16 * k, 16)] for k in range(K)]
        lo = ss_v[pl.ds(0, 16)][0]
        hi = ss_v[pl.ds(_SEG_W, 16)][0]

        zeros = jnp.zeros((16,), jnp.float32)
        ninf = jnp.full((16,), -jnp.inf, jnp.float32)

        for k in range(K):
            stage[pl.ds(16 * k, 16)] = zeros
            stage[pl.ds(D + 16 * k, 16)] = ninf
        for j in range(_SEG_W):
            pltpu.make_async_copy(
                stage.at[pl.ds(0, 2 * D)], out_hbm.at[seg0 + j], sem_o).start()
        for j in range(_SEG_W):
            pltpu.make_async_copy(
                stage.at[pl.ds(0, 2 * D)], out_hbm.at[seg0 + j], sem_o).wait()

        lo_al = (lo // 8) * 8
        n_chunks = jnp.maximum((hi - lo_al + _R - 1) // _R, 1)

        def x_start(t):
            return pl.multiple_of(jnp.minimum(lo_al + t * _R, N - _R), 8)

        def b_start(t):
            return pl.multiple_of(jnp.minimum(lo_al + t * _R, N - 320), 8)

        def x_dma(t):
            return pltpu.make_async_copy(
                x_hbm.at[pl.ds(x_start(t), _R)],
                xbuf.at[t % 2, pl.ds(0, _R)], sem_x)

        def b_dma(t):
            return pltpu.make_async_copy(
                b_hbm.at[pl.ds(b_start(t), 320)],
                bbuf.at[pl.ds((t % 2) * 320, 320)], sem_b)

        x_dma(0).start()
        b_dma(0).start()

        def out_wait():
            pltpu.make_async_copy(
                stage.at[pl.ds(0, 2 * D)], out_hbm.at[seg0], sem_o).wait()

        def chunk_body(t, carry):
            g0 = jnp.maximum(lo, lo_al + t * _R)
            g1 = jnp.minimum(lo_al + (t + 1) * _R, hi)
            start = x_start(t)
            bstart = b_start(t)
            x_dma(t).wait()
            b_dma(t).wait()

            @pl.when(t + 1 < n_chunks)
            def _():
                x_dma(t + 1).start()
                b_dma(t + 1).start()

            xs = xbuf.at[t % 2]
            bbase = (t % 2) * 320

            def score_of(bi):
                xk = [xs[bi, pl.ds(16 * k, 16)] for k in range(K)]
                pv = xk[0] * wv[0]
                for k in range(1, K):
                    pv = pv + xk[k] * wv[k]
                tlog = jnp.broadcast_to(jnp.sum(pv), (16,)) + bias_v
                sc = 1.0 / (1.0 + jnp.exp(-tlog))
                return xk, sc

            xk0, sc0 = score_of(g0 - start)

            def row_body(r, c):
                nflush = c[0]
                sc = c[1]
                xcur = c[2:2 + K]
                a = c[2 + K:]
                ids = bbuf[pl.ds(bbase + r - bstart, 16)]
                cur = ids[0]
                flush = (ids[1] != cur) | (r == hi - 1)
                sums = [a[k] + sc * xcur[k] for k in range(K)]
                maxs = [jnp.maximum(a[K + k], xcur[k]) for k in range(K)]
                xn, scn = score_of(r + 1 - start)

                @pl.when(flush)
                def _():
                    @pl.when(nflush >= 2)
                    def _():
                        out_wait()
                    sbase = pl.multiple_of((nflush % 2) * 2 * D, 8)
                    for k in range(K):
                        stage[pl.ds(sbase + 16 * k, 16)] = sums[k]
                        stage[pl.ds(sbase + D + 16 * k, 16)] = maxs[k]
                    pltpu.make_async_copy(
                        stage.at[pl.ds(sbase, 2 * D)],
                        out_hbm.at[cur], sem_o).start()

                out = (nflush + flush.astype(jnp.int32), scn) + tuple(xn)
                out += tuple(jnp.where(flush, zeros, sums[k])
                             for k in range(K))
                out += tuple(jnp.where(flush, ninf, maxs[k])
                             for k in range(K))
                return out

            c = (carry[0], sc0) + tuple(xk0) + tuple(carry[1:])
            res = lax.fori_loop(g0, g1, row_body, c)
            return (res[0],) + tuple(res[2 + K:])

        init = (jnp.int32(0),) + (zeros,) * K + (ninf,) * K
        fin = lax.fori_loop(0, n_chunks, chunk_body, init)
        nf = fin[0]

        @pl.when(nf >= 2)
        def _():
            out_wait()

        @pl.when(nf >= 1)
        def _():
            out_wait()

    return body(x, batch_i32, seg_starts, w, b)


def kernel(x, batch, W, b):
    N, D = x.shape
    bi32 = batch.astype(jnp.int32)
    ss = jnp.searchsorted(bi32, jnp.arange(_S + 1, dtype=jnp.int32),
                          side="left").astype(jnp.int32)
    ss_pad = jnp.concatenate(
        [ss, jnp.full((528 - _S - 1,), N, dtype=jnp.int32)])
    bp = jnp.full((16,), b[0], dtype=jnp.float32)
    return _sc_readout(x, bi32, ss_pad, W.reshape(D).astype(jnp.float32), bp)
